# baseline (speedup 1.0000x reference)
import functools

import jax
import jax.numpy as jnp
from jax import lax
from jax.experimental import pallas as pl
from jax.experimental.pallas import tpu as pltpu
from jax.experimental.pallas import tpu_sc as plsc

_T0 = (464, 1135, 2293, 3244)
_T1 = (318, 373, 547, 389, 1816)
_T2 = (922, 1263, 1310, 3621, 4950, 7932)

_B = 4
_S = 8192
_N = _B * _S


def _match(v, toks):
    m = v == toks[0]
    for t in toks[1:]:
        m = m | (v == t)
    return m


def kernel(doc_ids, w_match_0, w_match_1, w_match_2, w_seq):
    info = plsc.get_sparse_core_info()
    nw = info.num_cores * info.num_subcores
    L = info.num_lanes
    chunk = _N // nw

    mesh = plsc.VectorSubcoreMesh(core_axis_name="c", subcore_axis_name="s")

    @functools.partial(
        pl.kernel,
        mesh=mesh,
        out_type=jax.ShapeDtypeStruct((_N,), jnp.float32),
        scratch_types=[
            pltpu.VMEM((chunk + L,), jnp.int32),
            pltpu.VMEM((chunk,), jnp.float32),
            pltpu.VMEM((L,), jnp.float32),
            pltpu.VMEM((L,), jnp.float32),
            pltpu.VMEM((L,), jnp.float32),
            pltpu.VMEM((L,), jnp.float32),
        ],
    )
    def _seq_kernel(doc_hbm, w0_hbm, w1_hbm, w2_hbm, ws_hbm, out_hbm,
                    d, out_s, w0_v, w1_v, w2_v, ws_v):
        wid = lax.axis_index("s") * info.num_cores + lax.axis_index("c")
        base = pl.multiple_of(wid * chunk, chunk)
        halo_start = pl.multiple_of(jnp.maximum(base - L, 0), L)
        pltpu.sync_copy(doc_hbm.at[pl.ds(halo_start, L)], d.at[pl.ds(0, L)])
        pltpu.sync_copy(doc_hbm.at[pl.ds(base, chunk)], d.at[pl.ds(L, chunk)])
        pltpu.sync_copy(w0_hbm, w0_v)
        pltpu.sync_copy(w1_hbm, w1_v)
        pltpu.sync_copy(w2_hbm, w2_v)
        pltpu.sync_copy(ws_hbm, ws_v)
        w = w0_v[...] * w1_v[...] * w2_v[...] * ws_v[...]
        zero = jnp.zeros((L,), jnp.float32)
        base_in_row = lax.rem(base, _S)
        lane = lax.iota(jnp.int32, L)

        def body(j, carry):
            off = L + j * L
            v2 = d[pl.ds(off, L)]
            v1 = d[pl.ds(off - 1, L)]
            v0 = d[pl.ds(off - 2, L)]
            m = _match(v0, _T0) & _match(v1, _T1) & _match(v2, _T2)
            pos = base_in_row + j * L + lane
            valid = m & (pos >= 2)
            out_s[pl.ds(j * L, L)] = jnp.where(valid, w, zero)
            return carry

        lax.fori_loop(0, chunk // L, body, 0)
        pltpu.sync_copy(out_s, out_hbm.at[pl.ds(base, chunk)])

    out = _seq_kernel(
        doc_ids.reshape(_N),
        jnp.broadcast_to(w_match_0.astype(jnp.float32), (L,)),
        jnp.broadcast_to(w_match_1.astype(jnp.float32), (L,)),
        jnp.broadcast_to(w_match_2.astype(jnp.float32), (L,)),
        jnp.broadcast_to(w_seq.astype(jnp.float32), (L,)),
    )
    return out.reshape(_B, _S)


# speedup vs baseline: 1.1254x; 1.1254x over previous
import functools

import jax
import jax.numpy as jnp
from jax import lax
from jax.experimental import pallas as pl
from jax.experimental.pallas import tpu as pltpu
from jax.experimental.pallas import tpu_sc as plsc

_T0 = (464, 1135, 2293, 3244)
_T1 = (318, 373, 547, 389, 1816)
_T2 = (922, 1263, 1310, 3621, 4950, 7932)

_B = 4
_S = 8192
_N = _B * _S


def _match(v, toks):
    m = v == toks[0]
    for t in toks[1:]:
        m = m | (v == t)
    return m


def kernel(doc_ids, w_match_0, w_match_1, w_match_2, w_seq):
    info = plsc.get_sparse_core_info()
    nw = info.num_cores * info.num_subcores
    L = info.num_lanes
    chunk = _N // nw

    mesh = plsc.VectorSubcoreMesh(core_axis_name="c", subcore_axis_name="s")

    @functools.partial(
        pl.kernel,
        mesh=mesh,
        out_type=jax.ShapeDtypeStruct((_N,), jnp.float32),
        scratch_types=[
            pltpu.VMEM((chunk + L,), jnp.int32),
            pltpu.VMEM((chunk,), jnp.float32),
            pltpu.VMEM((4 * L,), jnp.float32),
            pltpu.SemaphoreType.DMA,
        ],
    )
    def _seq_kernel(doc_hbm, w_hbm, out_hbm, d, out_s, wv, sem):
        wid = lax.axis_index("s") * info.num_cores + lax.axis_index("c")
        base = pl.multiple_of(wid * chunk, chunk)
        halo_start = pl.multiple_of(jnp.maximum(base - L, 0), L)
        c_halo = pltpu.async_copy(
            doc_hbm.at[pl.ds(halo_start, L)], d.at[pl.ds(0, L)], sem)
        c_main = pltpu.async_copy(
            doc_hbm.at[pl.ds(base, chunk)], d.at[pl.ds(L, chunk)], sem)
        c_w = pltpu.async_copy(w_hbm, wv, sem)
        c_halo.wait()
        c_main.wait()
        c_w.wait()
        w = (wv[pl.ds(0, L)] * wv[pl.ds(L, L)]
             * wv[pl.ds(2 * L, L)] * wv[pl.ds(3 * L, L)])
        zero = jnp.zeros((L,), jnp.float32)
        base_in_row = lax.rem(base, _S)
        lane = lax.iota(jnp.int32, L)

        for j in range(chunk // L):
            off = L + j * L
            v2 = d[pl.ds(off, L)]
            v1 = d[pl.ds(off - 1, L)]
            v0 = d[pl.ds(off - 2, L)]
            m = _match(v0, _T0) & _match(v1, _T1) & _match(v2, _T2)
            if j == 0:
                m = m & (base_in_row + lane >= 2)
            out_s[pl.ds(j * L, L)] = jnp.where(m, w, zero)

        pltpu.sync_copy(out_s, out_hbm.at[pl.ds(base, chunk)])

    L16 = 16
    w_cat = jnp.concatenate([
        jnp.broadcast_to(w_match_0.astype(jnp.float32), (L16,)),
        jnp.broadcast_to(w_match_1.astype(jnp.float32), (L16,)),
        jnp.broadcast_to(w_match_2.astype(jnp.float32), (L16,)),
        jnp.broadcast_to(w_seq.astype(jnp.float32), (L16,)),
    ])
    out = _seq_kernel(doc_ids.reshape(_N), w_cat)
    return out.reshape(_B, _S)


# speedup vs baseline: 1.1797x; 1.0483x over previous
import functools

import jax
import jax.numpy as jnp
from jax import lax
from jax.experimental import pallas as pl
from jax.experimental.pallas import tpu as pltpu
from jax.experimental.pallas import tpu_sc as plsc

_T0 = (464, 1135, 2293, 3244)
_T1 = (318, 373, 547, 389, 1816)
_T2 = (922, 1263, 1310, 3621, 4950, 7932)

_B = 4
_S = 8192
_N = _B * _S


def _match(v, toks):
    m = v == toks[0]
    for t in toks[1:]:
        m = m | (v == t)
    return m


def kernel(doc_ids, w_match_0, w_match_1, w_match_2, w_seq):
    info = plsc.get_sparse_core_info()
    num_cores = 1
    nw = num_cores * info.num_subcores
    L = info.num_lanes
    chunk = _N // nw

    mesh = plsc.VectorSubcoreMesh(
        core_axis_name="c", subcore_axis_name="s", num_cores=num_cores)

    @functools.partial(
        pl.kernel,
        mesh=mesh,
        out_type=jax.ShapeDtypeStruct((_N,), jnp.float32),
        scratch_types=[
            pltpu.VMEM((chunk + L,), jnp.int32),
            pltpu.VMEM((chunk,), jnp.float32),
            pltpu.VMEM((4 * L,), jnp.float32),
            pltpu.SemaphoreType.DMA,
        ],
    )
    def _seq_kernel(doc_hbm, w_hbm, out_hbm, d, out_s, wv, sem):
        wid = lax.axis_index("s") * num_cores + lax.axis_index("c")
        base = pl.multiple_of(wid * chunk, chunk)
        halo_start = pl.multiple_of(jnp.maximum(base - L, 0), L)
        c_halo = pltpu.async_copy(
            doc_hbm.at[pl.ds(halo_start, L)], d.at[pl.ds(0, L)], sem)
        c_main = pltpu.async_copy(
            doc_hbm.at[pl.ds(base, chunk)], d.at[pl.ds(L, chunk)], sem)
        c_w = pltpu.async_copy(w_hbm, wv, sem)
        c_halo.wait()
        c_main.wait()
        c_w.wait()
        w = (wv[pl.ds(0, L)] * wv[pl.ds(L, L)]
             * wv[pl.ds(2 * L, L)] * wv[pl.ds(3 * L, L)])
        zero = jnp.zeros((L,), jnp.float32)
        base_in_row = lax.rem(base, _S)
        lane = lax.iota(jnp.int32, L)

        for j in range(chunk // L):
            off = L + j * L
            v2 = d[pl.ds(off, L)]
            v1 = d[pl.ds(off - 1, L)]
            v0 = d[pl.ds(off - 2, L)]
            m = _match(v0, _T0) & _match(v1, _T1) & _match(v2, _T2)
            if j == 0:
                m = m & (base_in_row + lane >= 2)
            out_s[pl.ds(j * L, L)] = jnp.where(m, w, zero)

        pltpu.sync_copy(out_s, out_hbm.at[pl.ds(base, chunk)])

    L16 = 16
    w_cat = jnp.concatenate([
        jnp.broadcast_to(w_match_0.astype(jnp.float32), (L16,)),
        jnp.broadcast_to(w_match_1.astype(jnp.float32), (L16,)),
        jnp.broadcast_to(w_match_2.astype(jnp.float32), (L16,)),
        jnp.broadcast_to(w_seq.astype(jnp.float32), (L16,)),
    ])
    out = _seq_kernel(doc_ids.reshape(_N), w_cat)
    return out.reshape(_B, _S)
